# Initial kernel scaffold; baseline (speedup 1.0000x reference)
import jax
import jax.numpy as jnp
from jax.experimental import pallas as pl


def kernel(x, h, edge_index, snorm_n, snorm_e, W1, W2, gamma, beta):
    raise NotImplementedError("write your pallas kernel here")


# speedup vs baseline: 4.0355x; 4.0355x over previous
import functools

import jax
import jax.numpy as jnp
from jax import lax
from jax.experimental import pallas as pl
from jax.experimental.pallas import tpu as pltpu
from jax.experimental.pallas import tpu_sc as plsc

NC = 2
NS = 16
NW = NC * NS


def _pq_body(x_ref, wa_ref, wb_ref, p_ref, q_ref):
    xv = x_ref[...]
    p_ref[...] = jnp.dot(xv, wa_ref[...], preferred_element_type=jnp.float32)
    q_ref[...] = jnp.dot(xv, wb_ref[...], preferred_element_type=jnp.float32)


def _hc_body(h_ref, wc_ref, o_ref):
    o_ref[...] = jnp.dot(h_ref[...], wc_ref[...],
                         preferred_element_type=jnp.float32)


def _out_body(m0_ref, m1_ref, w2t_ref, g_ref, b_ref, s_ref):
    m = m0_ref[0] + m1_ref[0]
    r = jnp.dot(m, w2t_ref[...], preferred_element_type=jnp.float32)
    mu = jnp.mean(r, axis=1, keepdims=True)
    var = jnp.mean((r - mu) * (r - mu), axis=1, keepdims=True)
    y = (r - mu) * lax.rsqrt(var + 1e-5) * g_ref[...] + b_ref[...]
    s_ref[...] = jnp.maximum(y, 0.0)


def _make_seg_kernel(NP, D, EW, CH, NCH, NR):
    mesh = plsc.VectorSubcoreMesh(core_axis_name="c", subcore_axis_name="s")

    @functools.partial(
        pl.kernel, mesh=mesh,
        out_type=jax.ShapeDtypeStruct((NC, NP, D), jnp.float32),
        scratch_types=[
            pltpu.VMEM((CH,), jnp.int32),
            pltpu.VMEM((CH,), jnp.int32),
            pltpu.VMEM((CH, D), jnp.float32),
            pltpu.VMEM((CH, D), jnp.float32),
            pltpu.VMEM((CH, D), jnp.float32),
            pltpu.VMEM_SHARED((NP, D), jnp.float32),
            pltpu.SemaphoreType.DMA,
            pltpu.SemaphoreType.DMA,
            pltpu.SemaphoreType.DMA,
        ],
    )
    def seg_kernel(p_hbm, q_hbm, hc_hbm, dstr_hbm, srcr_hbm, z_hbm, mpart_hbm,
                   dst_v, src_v, p_v, q_v, hc_v, m_sh, sem0, sem1, sem2):
        cid = lax.axis_index("c")
        sid = lax.axis_index("s")
        wid = cid * NS + sid
        pltpu.sync_copy(z_hbm, m_sh.at[pl.ds(sid * NR, NR)])
        plsc.subcore_barrier()
        base = wid * EW

        def chunk(j, carry):
            pltpu.sync_copy(dstr_hbm.at[wid, j], dst_v)
            pltpu.sync_copy(srcr_hbm.at[wid, j], src_v)
            cp = pltpu.async_copy(p_hbm.at[dst_v], p_v, sem0)
            cq = pltpu.async_copy(q_hbm.at[src_v], q_v, sem1)
            cc = pltpu.async_copy(hc_hbm.at[pl.ds(base + j * CH, CH)],
                                  hc_v, sem2)
            cp.wait()
            cq.wait()
            cc.wait()

            def row(i, carry2):
                for k in range(D // 16):
                    s = pl.ds(k * 16, 16)
                    v = p_v[i, s] + q_v[i, s] + hc_v[i, s]
                    hc_v[i, s] = jnp.maximum(v, 0.0)
                return carry2

            lax.fori_loop(0, CH, row, 0)
            pltpu.sync_copy(hc_v, m_sh.at[dst_v], add=True)
            return carry

        lax.fori_loop(0, NCH, chunk, 0)
        plsc.subcore_barrier()
        pltpu.sync_copy(m_sh.at[pl.ds(sid * NR, NR)],
                        mpart_hbm.at[cid, pl.ds(sid * NR, NR)])

    return seg_kernel


def _make_gather_kernel(N, D, E, EW, CH, NCH):
    mesh = plsc.VectorSubcoreMesh(core_axis_name="c", subcore_axis_name="s")

    @functools.partial(
        pl.kernel, mesh=mesh,
        out_type=jax.ShapeDtypeStruct((E, D), jnp.float32),
        scratch_types=[
            pltpu.VMEM((NCH, CH), jnp.int32),
            pltpu.VMEM((CH, D), jnp.float32),
            pltpu.SemaphoreType.DMA,
        ],
    )
    def gather_kernel(s_hbm, srcr_hbm, out_hbm, src_v, row_v, sem):
        cid = lax.axis_index("c")
        sid = lax.axis_index("s")
        wid = cid * NS + sid
        pltpu.sync_copy(srcr_hbm.at[wid], src_v)
        base = wid * EW

        def chunk(j, carry):
            pltpu.async_copy(s_hbm.at[src_v.at[j]], row_v, sem).wait()
            pltpu.sync_copy(row_v, out_hbm.at[pl.ds(base + j * CH, CH)])
            return carry

        lax.fori_loop(0, NCH, chunk, 0)

    return gather_kernel


def kernel(x, h, edge_index, snorm_n, snorm_e, W1, W2, gamma, beta):
    N, D = x.shape
    E = h.shape[0]
    assert E % NW == 0
    EW = E // NW
    CH = 80
    assert EW % CH == 0
    NCH = EW // CH
    NR = -(-N // NS)
    NR = -(-NR // 8) * 8
    NP = NR * NS

    src = edge_index[0]
    dst = edge_index[1]
    Wa = W1[:, :D].T
    Wb = W1[:, D:2 * D].T
    Wc = W1[:, 2 * D:].T
    W2t = W2.T

    P, Q = pl.pallas_call(
        _pq_body,
        out_shape=[jax.ShapeDtypeStruct((N, D), jnp.float32)] * 2,
    )(x, Wa, Wb)

    TE = 2000
    HC = pl.pallas_call(
        _hc_body,
        grid=(E // TE,),
        in_specs=[pl.BlockSpec((TE, D), lambda i: (i, 0)),
                  pl.BlockSpec((D, D), lambda i: (0, 0))],
        out_specs=pl.BlockSpec((TE, D), lambda i: (i, 0)),
        out_shape=jax.ShapeDtypeStruct((E, D), jnp.float32),
    )(h, Wc)

    dst_r = dst.reshape(NW, NCH, CH)
    src_r = src.reshape(NW, NCH, CH)
    zeros = jnp.zeros((NR, D), jnp.float32)
    seg = _make_seg_kernel(NP, D, EW, CH, NCH, NR)
    mpart = seg(P, Q, HC, dst_r, src_r, zeros)

    TN = 2000
    S = pl.pallas_call(
        _out_body,
        grid=(N // TN,),
        in_specs=[pl.BlockSpec((1, TN, D), lambda i: (0, i, 0)),
                  pl.BlockSpec((1, TN, D), lambda i: (1, i, 0)),
                  pl.BlockSpec((D, D), lambda i: (0, 0)),
                  pl.BlockSpec((1, D), lambda i: (0, 0)),
                  pl.BlockSpec((1, D), lambda i: (0, 0))],
        out_specs=pl.BlockSpec((TN, D), lambda i: (i, 0)),
        out_shape=jax.ShapeDtypeStruct((N, D), jnp.float32),
    )(mpart, mpart, W2t, gamma.reshape(1, D), beta.reshape(1, D))

    gather = _make_gather_kernel(N, D, E, EW, CH, NCH)
    return gather(S, src_r)


# speedup vs baseline: 5.4002x; 1.3382x over previous
import functools

import jax
import jax.numpy as jnp
from jax import lax
from jax.experimental import pallas as pl
from jax.experimental.pallas import tpu as pltpu
from jax.experimental.pallas import tpu_sc as plsc

NC = 2
NS = 16
NW = NC * NS


def _pq_body(x_ref, wa_ref, wb_ref, p_ref, q_ref):
    xv = x_ref[...]
    p_ref[...] = jnp.dot(xv, wa_ref[...], preferred_element_type=jnp.float32)
    q_ref[...] = jnp.dot(xv, wb_ref[...], preferred_element_type=jnp.float32)


def _hc_body(h_ref, wc_ref, o_ref):
    o_ref[...] = jnp.dot(h_ref[...], wc_ref[...],
                         preferred_element_type=jnp.float32)


def _out_body(m0_ref, m1_ref, w2t_ref, g_ref, b_ref, s_ref):
    m = m0_ref[0] + m1_ref[0]
    r = jnp.dot(m, w2t_ref[...], preferred_element_type=jnp.float32)
    mu = jnp.mean(r, axis=1, keepdims=True)
    var = jnp.mean((r - mu) * (r - mu), axis=1, keepdims=True)
    y = (r - mu) * lax.rsqrt(var + 1e-5) * g_ref[...] + b_ref[...]
    s_ref[...] = jnp.maximum(y, 0.0)


def _make_seg_kernel(NP, D, EW, CH, NCH, NR):
    mesh = plsc.VectorSubcoreMesh(core_axis_name="c", subcore_axis_name="s")
    T = NCH // 2

    @functools.partial(
        pl.kernel, mesh=mesh,
        out_type=jax.ShapeDtypeStruct((NC, NP, D), jnp.float32),
        scratch_types=[
            pltpu.VMEM((2, CH), jnp.int32),
            pltpu.VMEM((2, CH), jnp.int32),
            pltpu.VMEM((CH, D), jnp.float32),
            pltpu.VMEM((CH, D), jnp.float32),
            pltpu.VMEM((CH, D), jnp.float32),
            pltpu.VMEM((CH, D), jnp.float32),
            pltpu.VMEM((CH, D), jnp.float32),
            pltpu.VMEM((CH, D), jnp.float32),
            pltpu.VMEM_SHARED((NP, D), jnp.float32),
            pltpu.SemaphoreType.DMA,
            pltpu.SemaphoreType.DMA,
        ],
    )
    def seg_kernel(p_hbm, q_hbm, hc_hbm, idxc_hbm, z_hbm, mpart_hbm,
                   ix0, ix1, p0, p1, q0, q1, h0, h1, m_sh, gs0, gs1):
        cid = lax.axis_index("c")
        sid = lax.axis_index("s")
        wid = cid * NS + sid
        base = wid * EW

        def stage_idx(j, ix):
            pltpu.sync_copy(idxc_hbm.at[wid, j], ix)

        def fire_g(j, ix, pb, qb, hb, gsem):
            pltpu.async_copy(p_hbm.at[ix.at[0]], pb, gsem)
            pltpu.async_copy(q_hbm.at[ix.at[1]], qb, gsem)
            pltpu.async_copy(hc_hbm.at[pl.ds(base + j * CH, CH)], hb, gsem)

        def process(ix, pb, qb, hb, gsem):
            pltpu.make_async_copy(p_hbm.at[ix.at[0]], pb, gsem).wait()
            pltpu.make_async_copy(q_hbm.at[ix.at[1]], qb, gsem).wait()
            pltpu.make_async_copy(hc_hbm.at[pl.ds(base, CH)], hb, gsem).wait()

            def row(i, carry2):
                for k in range(D // 16):
                    s = pl.ds(k * 16, 16)
                    v = pb[i, s] + qb[i, s] + hb[i, s]
                    hb[i, s] = jnp.maximum(v, 0.0)
                return carry2

            lax.fori_loop(0, CH, row, 0)
            pltpu.sync_copy(hb, m_sh.at[ix.at[0]], add=True)

        pltpu.sync_copy(z_hbm, m_sh.at[pl.ds(sid * NR, NR)])
        stage_idx(0, ix0)
        fire_g(0, ix0, p0, q0, h0, gs0)
        plsc.subcore_barrier()

        def pair(t, carry):
            j = 2 * t
            stage_idx(j + 1, ix1)
            fire_g(j + 1, ix1, p1, q1, h1, gs1)
            process(ix0, p0, q0, h0, gs0)
            stage_idx(j + 2, ix0)
            fire_g(j + 2, ix0, p0, q0, h0, gs0)
            process(ix1, p1, q1, h1, gs1)
            return carry

        lax.fori_loop(0, T - 1, pair, 0)
        stage_idx(NCH - 1, ix1)
        fire_g(NCH - 1, ix1, p1, q1, h1, gs1)
        process(ix0, p0, q0, h0, gs0)
        process(ix1, p1, q1, h1, gs1)
        plsc.subcore_barrier()
        pltpu.sync_copy(m_sh.at[pl.ds(sid * NR, NR)],
                        mpart_hbm.at[cid, pl.ds(sid * NR, NR)])

    return seg_kernel


def _make_gather_kernel(N, D, E, EW, CH, NCH):
    mesh = plsc.VectorSubcoreMesh(core_axis_name="c", subcore_axis_name="s")
    T = (NCH - 1) // 2

    @functools.partial(
        pl.kernel, mesh=mesh,
        out_type=jax.ShapeDtypeStruct((E, D), jnp.float32),
        scratch_types=[
            pltpu.VMEM((NCH, CH), jnp.int32),
            pltpu.VMEM((CH, D), jnp.float32),
            pltpu.VMEM((CH, D), jnp.float32),
            pltpu.SemaphoreType.DMA,
            pltpu.SemaphoreType.DMA,
        ],
    )
    def gather_kernel(s_hbm, srcr_hbm, out_hbm, src_v, r0, r1, gs0, gs1):
        cid = lax.axis_index("c")
        sid = lax.axis_index("s")
        wid = cid * NS + sid
        base = wid * EW

        def fire_g(j, rb, gsem):
            pltpu.async_copy(s_hbm.at[src_v.at[j]], rb, gsem)

        def wait_g(rb, gsem):
            pltpu.make_async_copy(s_hbm.at[src_v.at[0]], rb, gsem).wait()

        def write(j, rb):
            pltpu.sync_copy(rb, out_hbm.at[pl.ds(base + j * CH, CH)])

        pltpu.sync_copy(srcr_hbm.at[wid], src_v)
        fire_g(0, r0, gs0)

        def pair(t, carry):
            j = 2 * t
            fire_g(j + 1, r1, gs1)
            wait_g(r0, gs0)
            write(j, r0)
            fire_g(j + 2, r0, gs0)
            wait_g(r1, gs1)
            write(j + 1, r1)
            return carry

        lax.fori_loop(0, T, pair, 0)
        wait_g(r0, gs0)
        write(NCH - 1, r0)

    return gather_kernel


def kernel(x, h, edge_index, snorm_n, snorm_e, W1, W2, gamma, beta):
    N, D = x.shape
    E = h.shape[0]
    assert E % NW == 0
    EW = E // NW
    CH = 40
    assert EW % CH == 0 and (EW // CH) % 2 == 0
    NCH = EW // CH
    CH2 = 80
    assert EW % CH2 == 0
    NCH2 = EW // CH2
    assert NCH2 % 2 == 1
    NR = -(-N // NS)
    NR = -(-NR // 8) * 8
    NP = NR * NS

    src = edge_index[0]
    dst = edge_index[1]
    Wa = W1[:, :D].T
    Wb = W1[:, D:2 * D].T
    Wc = W1[:, 2 * D:].T
    W2t = W2.T

    P, Q = pl.pallas_call(
        _pq_body,
        out_shape=[jax.ShapeDtypeStruct((N, D), jnp.float32)] * 2,
    )(x, Wa, Wb)

    TE = 2000
    HC = pl.pallas_call(
        _hc_body,
        grid=(E // TE,),
        in_specs=[pl.BlockSpec((TE, D), lambda i: (i, 0)),
                  pl.BlockSpec((D, D), lambda i: (0, 0))],
        out_specs=pl.BlockSpec((TE, D), lambda i: (i, 0)),
        out_shape=jax.ShapeDtypeStruct((E, D), jnp.float32),
    )(h, Wc)

    idx_comb = jnp.stack([dst.reshape(NW, NCH, CH),
                          src.reshape(NW, NCH, CH)], axis=2)
    zeros = jnp.zeros((NR, D), jnp.float32)
    seg = _make_seg_kernel(NP, D, EW, CH, NCH, NR)
    mpart = seg(P, Q, HC, idx_comb, zeros)

    TN = 2000
    S = pl.pallas_call(
        _out_body,
        grid=(N // TN,),
        in_specs=[pl.BlockSpec((1, TN, D), lambda i: (0, i, 0)),
                  pl.BlockSpec((1, TN, D), lambda i: (1, i, 0)),
                  pl.BlockSpec((D, D), lambda i: (0, 0)),
                  pl.BlockSpec((1, D), lambda i: (0, 0)),
                  pl.BlockSpec((1, D), lambda i: (0, 0))],
        out_specs=pl.BlockSpec((TN, D), lambda i: (i, 0)),
        out_shape=jax.ShapeDtypeStruct((N, D), jnp.float32),
    )(mpart, mpart, W2t, gamma.reshape(1, D), beta.reshape(1, D))

    src_r2 = src.reshape(NW, NCH2, CH2)
    gather = _make_gather_kernel(N, D, E, EW, CH2, NCH2)
    return gather(S, src_r2)


# speedup vs baseline: 5.4524x; 1.0097x over previous
import functools

import jax
import jax.numpy as jnp
from jax import lax
from jax.experimental import pallas as pl
from jax.experimental.pallas import tpu as pltpu
from jax.experimental.pallas import tpu_sc as plsc

NC = 2
NS = 16
NW = NC * NS


def _pq_body(x_ref, wa_ref, wb_ref, p_ref, q_ref):
    xv = x_ref[...]
    p_ref[...] = jnp.dot(xv, wa_ref[...], preferred_element_type=jnp.float32)
    q_ref[...] = jnp.dot(xv, wb_ref[...], preferred_element_type=jnp.float32)


def _hc_body(h_ref, wc_ref, o_ref):
    o_ref[...] = jnp.dot(h_ref[...], wc_ref[...],
                         preferred_element_type=jnp.float32)


def _out_body(m0_ref, m1_ref, w2t_ref, g_ref, b_ref, s_ref):
    m = m0_ref[0] + m1_ref[0]
    r = jnp.dot(m, w2t_ref[...], preferred_element_type=jnp.float32)
    mu = jnp.mean(r, axis=1, keepdims=True)
    var = jnp.mean((r - mu) * (r - mu), axis=1, keepdims=True)
    y = (r - mu) * lax.rsqrt(var + 1e-5) * g_ref[...] + b_ref[...]
    s_ref[...] = jnp.maximum(y, 0.0)


def _make_seg_kernel(NP, D, EW, CH, NCH, NR):
    mesh = plsc.VectorSubcoreMesh(core_axis_name="c", subcore_axis_name="s")
    T = NCH // 2

    @functools.partial(
        pl.kernel, mesh=mesh,
        out_type=jax.ShapeDtypeStruct((NC, NP, D), jnp.float32),
        scratch_types=[
            pltpu.VMEM((2, CH), jnp.int32),
            pltpu.VMEM((2, CH), jnp.int32),
            pltpu.VMEM((CH, D), jnp.float32),
            pltpu.VMEM((CH, D), jnp.float32),
            pltpu.VMEM((CH, D), jnp.float32),
            pltpu.VMEM((CH, D), jnp.float32),
            pltpu.VMEM((CH, D), jnp.float32),
            pltpu.VMEM((CH, D), jnp.float32),
            pltpu.VMEM_SHARED((NP, D), jnp.float32),
            pltpu.SemaphoreType.DMA,
            pltpu.SemaphoreType.DMA,
        ],
    )
    def seg_kernel(p_hbm, q_hbm, hc_hbm, idxc_hbm, z_hbm, mpart_hbm,
                   ix0, ix1, p0, p1, q0, q1, h0, h1, m_sh, gs0, gs1):
        cid = lax.axis_index("c")
        sid = lax.axis_index("s")
        wid = cid * NS + sid
        base = wid * EW

        def stage_idx(j, ix):
            pltpu.sync_copy(idxc_hbm.at[wid, j], ix)

        def fire_g(j, ix, pb, qb, hb, gsem):
            pltpu.async_copy(p_hbm.at[ix.at[0]], pb, gsem)
            pltpu.async_copy(q_hbm.at[ix.at[1]], qb, gsem)
            pltpu.async_copy(hc_hbm.at[pl.ds(base + j * CH, CH)], hb, gsem)

        def process(ix, pb, qb, hb, gsem):
            pltpu.make_async_copy(p_hbm.at[ix.at[0]], pb, gsem).wait()
            pltpu.make_async_copy(q_hbm.at[ix.at[1]], qb, gsem).wait()
            pltpu.make_async_copy(hc_hbm.at[pl.ds(base, CH)], hb, gsem).wait()

            @plsc.parallel_loop(0, CH)
            def row(i):
                for k in range(D // 16):
                    s = pl.ds(k * 16, 16)
                    v = pb[i, s] + qb[i, s] + hb[i, s]
                    hb[i, s] = jnp.maximum(v, 0.0)
            pltpu.sync_copy(hb, m_sh.at[ix.at[0]], add=True)

        pltpu.sync_copy(z_hbm, m_sh.at[pl.ds(sid * NR, NR)])
        stage_idx(0, ix0)
        fire_g(0, ix0, p0, q0, h0, gs0)
        plsc.subcore_barrier()

        def pair(t, carry):
            j = 2 * t
            stage_idx(j + 1, ix1)
            fire_g(j + 1, ix1, p1, q1, h1, gs1)
            process(ix0, p0, q0, h0, gs0)
            stage_idx(j + 2, ix0)
            fire_g(j + 2, ix0, p0, q0, h0, gs0)
            process(ix1, p1, q1, h1, gs1)
            return carry

        lax.fori_loop(0, T - 1, pair, 0)
        stage_idx(NCH - 1, ix1)
        fire_g(NCH - 1, ix1, p1, q1, h1, gs1)
        process(ix0, p0, q0, h0, gs0)
        process(ix1, p1, q1, h1, gs1)
        plsc.subcore_barrier()
        pltpu.sync_copy(m_sh.at[pl.ds(sid * NR, NR)],
                        mpart_hbm.at[cid, pl.ds(sid * NR, NR)])

    return seg_kernel


def _make_gather_kernel(N, D, E, EW, CH, NCH):
    mesh = plsc.VectorSubcoreMesh(core_axis_name="c", subcore_axis_name="s")
    assert NCH >= 9 and (NCH - 5) % 4 == 0
    TQ = (NCH - 5) // 4

    @functools.partial(
        pl.kernel, mesh=mesh,
        out_type=jax.ShapeDtypeStruct((E, D), jnp.float32),
        scratch_types=[
            pltpu.VMEM((NCH, CH), jnp.int32),
            pltpu.VMEM((CH, D), jnp.float32),
            pltpu.VMEM((CH, D), jnp.float32),
            pltpu.VMEM((CH, D), jnp.float32),
            pltpu.VMEM((CH, D), jnp.float32),
            pltpu.SemaphoreType.DMA,
            pltpu.SemaphoreType.DMA,
            pltpu.SemaphoreType.DMA,
            pltpu.SemaphoreType.DMA,
            pltpu.SemaphoreType.DMA,
            pltpu.SemaphoreType.DMA,
            pltpu.SemaphoreType.DMA,
            pltpu.SemaphoreType.DMA,
        ],
    )
    def gather_kernel(s_hbm, srcr_hbm, out_hbm, src_v, r0, r1, r2, r3,
                      g0, g1, g2, g3, o0, o1, o2, o3):
        cid = lax.axis_index("c")
        sid = lax.axis_index("s")
        wid = cid * NS + sid
        base = wid * EW
        R = (r0, r1, r2, r3)
        GS = (g0, g1, g2, g3)
        OS = (o0, o1, o2, o3)

        def fire_g(j, b):
            pltpu.async_copy(s_hbm.at[src_v.at[j]], R[b], GS[b])

        def wait_g(b):
            pltpu.make_async_copy(s_hbm.at[src_v.at[0]], R[b], GS[b]).wait()

        def fire_w(j, b):
            pltpu.async_copy(R[b], out_hbm.at[pl.ds(base + j * CH, CH)],
                             OS[b])

        def wait_w(b):
            pltpu.make_async_copy(R[b], out_hbm.at[pl.ds(base, CH)],
                                  OS[b]).wait()

        pltpu.sync_copy(srcr_hbm.at[wid], src_v)
        fire_g(0, 0)
        fire_g(1, 1)
        wait_g(0)
        fire_w(0, 0)
        fire_g(2, 2)
        wait_g(1)
        fire_w(1, 1)
        fire_g(3, 3)

        def step(j, b):
            wait_g(b)
            fire_w(j, b)
            nb = (b + 2) % 4
            wait_w(nb)
            fire_g(j + 2, nb)

        def quad(t, carry):
            j0 = 2 + 4 * t
            for b4 in range(4):
                step(j0 + b4, (2 + b4) % 4)
            return carry

        lax.fori_loop(0, TQ, quad, 0)
        j = NCH - 3
        wait_g(2)
        fire_w(j, 2)
        wait_w(0)
        fire_g(j + 2, 0)
        wait_g(3)
        fire_w(j + 1, 3)
        wait_g(0)
        fire_w(j + 2, 0)
        wait_w(1)
        wait_w(2)
        wait_w(3)
        wait_w(0)

    return gather_kernel


def kernel(x, h, edge_index, snorm_n, snorm_e, W1, W2, gamma, beta):
    N, D = x.shape
    E = h.shape[0]
    assert E % NW == 0
    EW = E // NW
    CH = 40
    assert EW % CH == 0 and (EW // CH) % 2 == 0
    NCH = EW // CH
    CH2 = 80
    assert EW % CH2 == 0
    NCH2 = EW // CH2
    assert NCH2 % 2 == 1
    NR = -(-N // NS)
    NR = -(-NR // 8) * 8
    NP = NR * NS

    src = edge_index[0]
    dst = edge_index[1]
    Wa = W1[:, :D].T
    Wb = W1[:, D:2 * D].T
    Wc = W1[:, 2 * D:].T
    W2t = W2.T

    P, Q = pl.pallas_call(
        _pq_body,
        out_shape=[jax.ShapeDtypeStruct((N, D), jnp.float32)] * 2,
    )(x, Wa, Wb)

    TE = 2000
    HC = pl.pallas_call(
        _hc_body,
        grid=(E // TE,),
        in_specs=[pl.BlockSpec((TE, D), lambda i: (i, 0)),
                  pl.BlockSpec((D, D), lambda i: (0, 0))],
        out_specs=pl.BlockSpec((TE, D), lambda i: (i, 0)),
        out_shape=jax.ShapeDtypeStruct((E, D), jnp.float32),
    )(h, Wc)

    idx_comb = jnp.stack([dst.reshape(NW, NCH, CH),
                          src.reshape(NW, NCH, CH)], axis=2)
    zeros = jnp.zeros((NR, D), jnp.float32)
    seg = _make_seg_kernel(NP, D, EW, CH, NCH, NR)
    mpart = seg(P, Q, HC, idx_comb, zeros)

    TN = 2000
    S = pl.pallas_call(
        _out_body,
        grid=(N // TN,),
        in_specs=[pl.BlockSpec((1, TN, D), lambda i: (0, i, 0)),
                  pl.BlockSpec((1, TN, D), lambda i: (1, i, 0)),
                  pl.BlockSpec((D, D), lambda i: (0, 0)),
                  pl.BlockSpec((1, D), lambda i: (0, 0)),
                  pl.BlockSpec((1, D), lambda i: (0, 0))],
        out_specs=pl.BlockSpec((TN, D), lambda i: (i, 0)),
        out_shape=jax.ShapeDtypeStruct((N, D), jnp.float32),
    )(mpart, mpart, W2t, gamma.reshape(1, D), beta.reshape(1, D))

    src_r2 = src.reshape(NW, NCH2, CH2)
    gather = _make_gather_kernel(N, D, E, EW, CH2, NCH2)
    return gather(S, src_r2)
